# baseline (speedup 1.0000x reference)
import functools

import jax
import jax.numpy as jnp
from jax import lax
from jax.experimental import pallas as pl
from jax.experimental.pallas import tpu as pltpu
from jax.experimental.pallas import tpu_sc as plsc

F32 = jnp.float32
I32 = jnp.int32

H = 128
OUT = 10
NUM_LAYERS = 4
N_MOL = 50000
N_REACT = 10000
E = 320000

NC = 2
NS = 16
NW = NC * NS

NM_P = 51200
NR_P = 10240
CHUNK = 128
E_PAD = 327680
EROWS = E_PAD // CHUNK
TCH = EROWS // NW

R_STRIPE = NR_P // NS
M_STRIPE = NM_P // NS

RING_R = 4
RING_M = 8

_SC_MESH = plsc.VectorSubcoreMesh(core_axis_name="c", subcore_axis_name="s")
_SC_PARAMS = pltpu.CompilerParams(use_tc_tiling_on_sc=False)


def _worker_base():
    c = lax.axis_index("c")
    s = lax.axis_index("s")
    return c, s, (s * NC + c) * TCH


def _ring_pass(table, src_v, dst_v, bufs, sems, acc):
    depth = len(bufs)

    def fire(j, k):
        return pltpu.async_copy(table.at[src_v.at[j]], bufs[k], sems[k])

    for k in range(depth):
        fire(k, k)

    def group(g, carry):
        for k in range(depth):
            j = g * depth + k
            pltpu.make_async_copy(table.at[src_v.at[j]], bufs[k], sems[k]).wait()
            pltpu.sync_copy(bufs[k], acc.at[dst_v.at[j]], add=True)

            @pl.when(j + depth < TCH)
            def _():
                fire(j + depth, k)
        return carry

    lax.fori_loop(0, TCH // depth, group, 0)


def _m2r_body(with_cnt, xmt, srcs_hbm, dst_hbm, zr_hbm, *rest):
    if with_cnt:
        (ones_hbm, zc_hbm, out_hbm, cnt_hbm, src_v, dst_v,
         b0, b1, b2, b3, ones_v, acc, cacc, s0, s1, s2, s3, cs) = rest
    else:
        (out_hbm, src_v, dst_v, b0, b1, b2, b3, acc, s0, s1, s2, s3) = rest
    bufs = (b0, b1, b2, b3)
    sems = (s0, s1, s2, s3)
    c, s, base = _worker_base()
    r0 = s * R_STRIPE
    pltpu.sync_copy(dst_hbm.at[pl.ds(base, TCH)], dst_v)
    if with_cnt:
        pltpu.sync_copy(ones_hbm, ones_v)
        pltpu.sync_copy(zc_hbm, cacc.at[pl.ds(r0, R_STRIPE)])
    for p in range(2):
        pltpu.sync_copy(srcs_hbm.at[p, pl.ds(base, TCH)], src_v)
        pltpu.sync_copy(zr_hbm, acc.at[pl.ds(r0, R_STRIPE)])
        plsc.subcore_barrier()
        _ring_pass(xmt, src_v, dst_v, bufs, sems, acc)
        if with_cnt and p == 0:
            def cfire(g, carry):
                for k in range(4):
                    j = g * 4 + k
                    pltpu.async_copy(ones_v, cacc.at[dst_v.at[j]], cs, add=True)
                return carry

            lax.fori_loop(0, TCH // 4, cfire, 0)

            def cdrain(g, carry):
                for k in range(4):
                    j = g * 4 + k
                    pltpu.make_async_copy(
                        ones_v, cacc.at[dst_v.at[j]], cs).wait()
                return carry

            lax.fori_loop(0, TCH // 4, cdrain, 0)
        plsc.subcore_barrier()
        pltpu.sync_copy(acc.at[pl.ds(r0, R_STRIPE)],
                        out_hbm.at[c, p, pl.ds(r0, R_STRIPE)])
    if with_cnt:
        pltpu.sync_copy(cacc.at[pl.ds(r0, R_STRIPE)],
                        cnt_hbm.at[c, pl.ds(r0, R_STRIPE)])


_m2r_scratch = [
    pltpu.VMEM((TCH, CHUNK), I32),
    pltpu.VMEM((TCH, CHUNK), I32),
    pltpu.VMEM((CHUNK, 64), F32),
    pltpu.VMEM((CHUNK, 64), F32),
    pltpu.VMEM((CHUNK, 64), F32),
    pltpu.VMEM((CHUNK, 64), F32),
]

_sc_m2r_cnt = functools.partial(
    pl.kernel,
    out_type=(jax.ShapeDtypeStruct((NC, 2, NR_P, 64), F32),
              jax.ShapeDtypeStruct((NC, NR_P, 16), F32)),
    mesh=_SC_MESH,
    compiler_params=_SC_PARAMS,
    scratch_types=_m2r_scratch[:2] + _m2r_scratch[2:] + [
        pltpu.VMEM((CHUNK, 16), F32),
        pltpu.VMEM_SHARED((NR_P, 64), F32),
        pltpu.VMEM_SHARED((NR_P, 16), F32),
    ] + [pltpu.SemaphoreType.DMA] * 5,
)(functools.partial(_m2r_body, True))


_sc_m2r = functools.partial(
    pl.kernel,
    out_type=jax.ShapeDtypeStruct((NC, 2, NR_P, 64), F32),
    mesh=_SC_MESH,
    compiler_params=_SC_PARAMS,
    scratch_types=_m2r_scratch + [
        pltpu.VMEM_SHARED((NR_P, 64), F32),
    ] + [pltpu.SemaphoreType.DMA] * 4,
)(functools.partial(_m2r_body, False))


@functools.partial(
    pl.kernel,
    out_type=jax.ShapeDtypeStruct((NC, 8, NM_P, 16), F32),
    mesh=_SC_MESH,
    compiler_params=_SC_PARAMS,
    scratch_types=[
        pltpu.VMEM((TCH, CHUNK), I32),
        pltpu.VMEM((TCH, CHUNK), I32),
    ] + [pltpu.VMEM((CHUNK, 16), F32)] * RING_M + [
        pltpu.VMEM_SHARED((NM_P, 16), F32),
    ] + [pltpu.SemaphoreType.DMA] * RING_M,
)
def _sc_r2m(xrt, srcs_hbm, dst_hbm, zm_hbm, out_hbm, src_v, dst_v, *rest):
    bufs = rest[:RING_M]
    acc = rest[RING_M]
    sems = rest[RING_M + 1:]
    c, s, base = _worker_base()
    r0 = s * M_STRIPE
    pltpu.sync_copy(dst_hbm.at[pl.ds(base, TCH)], dst_v)
    for p in range(8):
        pltpu.sync_copy(srcs_hbm.at[p, pl.ds(base, TCH)], src_v)
        pltpu.sync_copy(zm_hbm, acc.at[pl.ds(r0, M_STRIPE)])
        plsc.subcore_barrier()
        _ring_pass(xrt, src_v, dst_v, bufs, sems, acc)
        plsc.subcore_barrier()
        pltpu.sync_copy(acc.at[pl.ds(r0, M_STRIPE)],
                        out_hbm.at[c, p, pl.ds(r0, M_STRIPE)])


@functools.partial(
    pl.kernel,
    out_type=jax.ShapeDtypeStruct((NC, NM_P, 16), F32),
    mesh=_SC_MESH,
    compiler_params=_SC_PARAMS,
    scratch_types=[
        pltpu.VMEM((TCH, CHUNK), I32),
        pltpu.VMEM((CHUNK, 16), F32),
        pltpu.VMEM_SHARED((NM_P, 16), F32),
        pltpu.SemaphoreType.DMA,
    ],
)
def _sc_cnt_m(dst_hbm, ones_hbm, zc_hbm, out_hbm, dst_v, ones_v, acc, s0):
    c, s, base = _worker_base()
    r0 = s * M_STRIPE
    pltpu.sync_copy(dst_hbm.at[pl.ds(base, TCH)], dst_v)
    pltpu.sync_copy(ones_hbm, ones_v)
    pltpu.sync_copy(zc_hbm, acc.at[pl.ds(r0, M_STRIPE)])
    plsc.subcore_barrier()

    def cfire(g, carry):
        for k in range(4):
            j = g * 4 + k
            pltpu.async_copy(ones_v, acc.at[dst_v.at[j]], s0, add=True)
        return carry

    lax.fori_loop(0, TCH // 4, cfire, 0)

    def cdrain(g, carry):
        for k in range(4):
            j = g * 4 + k
            pltpu.make_async_copy(ones_v, acc.at[dst_v.at[j]], s0).wait()
        return carry

    lax.fori_loop(0, TCH // 4, cdrain, 0)
    plsc.subcore_barrier()
    pltpu.sync_copy(acc.at[pl.ds(r0, M_STRIPE)],
                    out_hbm.at[c, pl.ds(r0, M_STRIPE)])


def _dot_t(a, b):
    return lax.dot_general(a, b, (((1,), (1,)), ((), ())),
                           preferred_element_type=F32)


def _dense_r_body(final, agg_ref, cnt_ref, x_ref, wl_ref, bl_ref, wr_ref,
                  *rest):
    if final:
        wo_ref, bo_ref, o_ref = rest
    else:
        (o_ref,) = rest
    parts = [agg_ref[0, p] + agg_ref[1, p] for p in range(2)]
    ssum = jnp.concatenate(parts, axis=1)
    cnt = cnt_ref[0][:, 0:1] + cnt_ref[1][:, 0:1]
    agg = ssum * (1.0 / jnp.maximum(cnt, 1.0))
    h = _dot_t(agg, wl_ref[...]) + bl_ref[...] + _dot_t(x_ref[...], wr_ref[...])
    r = jnp.maximum(h, 0.0)
    if final:
        o_ref[...] = _dot_t(r, wo_ref[...]) + bo_ref[...]
    else:
        o_ref[...] = r


def _make_dense_r(final):
    blk = 1024
    grid = NR_P // blk
    full = lambda shape: pl.BlockSpec(shape, lambda i: (0,) * len(shape))
    in_specs = [
        pl.BlockSpec((NC, 2, blk, 64), lambda i: (0, 0, i, 0)),
        pl.BlockSpec((NC, blk, 16), lambda i: (0, i, 0)),
        pl.BlockSpec((blk, H), lambda i: (i, 0)),
        full((H, H)), full((1, H)), full((H, H)),
    ]
    if final:
        in_specs += [full((H, H)), full((1, H))]
    return pl.pallas_call(
        functools.partial(_dense_r_body, final),
        grid=(grid,),
        in_specs=in_specs,
        out_specs=pl.BlockSpec((blk, H), lambda i: (i, 0)),
        out_shape=jax.ShapeDtypeStruct((NR_P, H), F32),
    )


def _dense_m_body(agg_ref, cnt_ref, x_ref, wl_ref, bl_ref, wr_ref, o_ref):
    parts = [agg_ref[0, p] + agg_ref[1, p] for p in range(8)]
    ssum = jnp.concatenate(parts, axis=1)
    cnt = cnt_ref[0][:, 0:1] + cnt_ref[1][:, 0:1]
    agg = ssum * (1.0 / jnp.maximum(cnt, 1.0))
    h = _dot_t(agg, wl_ref[...]) + bl_ref[...] + _dot_t(x_ref[...], wr_ref[...])
    o_ref[...] = jnp.maximum(h, 0.0)


def _make_dense_m():
    blk = 1024
    grid = NM_P // blk
    full = lambda shape: pl.BlockSpec(shape, lambda i: (0,) * len(shape))
    return pl.pallas_call(
        _dense_m_body,
        grid=(grid,),
        in_specs=[
            pl.BlockSpec((NC, 8, blk, 16), lambda i: (0, 0, i, 0)),
            pl.BlockSpec((NC, blk, 16), lambda i: (0, i, 0)),
            pl.BlockSpec((blk, H), lambda i: (i, 0)),
            full((H, H)), full((1, H)), full((H, H)),
        ],
        out_specs=pl.BlockSpec((blk, H), lambda i: (i, 0)),
        out_shape=jax.ShapeDtypeStruct((NM_P, H), F32),
    )


_dense_r = _make_dense_r(False)
_dense_r_final = _make_dense_r(True)
_dense_m = _make_dense_m()


def _pad_edges(row, fill):
    row = row.astype(I32)
    return jnp.concatenate(
        [row, jnp.full((E_PAD - E,), fill, dtype=I32)]).reshape(EROWS, CHUNK)


def _split_cols(x, width):
    n, d = x.shape
    k = d // width
    return x.reshape(n, k, width).transpose(1, 0, 2).reshape(k * n, width)


def kernel(x_molecule, x_reaction, edge_index_m2r, edge_index_r2m, params):
    xm = jnp.pad(x_molecule, ((0, NM_P - N_MOL), (0, 0)))
    xr = jnp.pad(x_reaction, ((0, NR_P - N_REACT), (0, 0)))

    src_m2r = _pad_edges(edge_index_m2r[0], 0)
    dst_m2r = _pad_edges(edge_index_m2r[1], N_REACT)
    dst_r2m = _pad_edges(edge_index_r2m[1], N_MOL)
    src_r2m = _pad_edges(edge_index_r2m[0], 0)
    srcs_m2r = jnp.stack([src_m2r + p * NM_P for p in range(2)])
    srcs_r2m = jnp.stack([src_r2m + p * NR_P for p in range(8)])

    ones16 = jnp.ones((CHUNK, 16), F32)
    z_r = jnp.zeros((R_STRIPE, 64), F32)
    z_rc = jnp.zeros((R_STRIPE, 16), F32)
    z_m = jnp.zeros((M_STRIPE, 16), F32)
    z_mc = jnp.zeros((M_STRIPE, 16), F32)

    cnt_m = _sc_cnt_m(dst_r2m, ones16, z_mc)

    cnt_r = None
    for l in range(NUM_LAYERS):
        xmt = _split_cols(xm, 64)
        if l == 0:
            agg_r, cnt_r = _sc_m2r_cnt(xmt, srcs_m2r, dst_m2r, z_r,
                                       ones16, z_rc)
        else:
            agg_r = _sc_m2r(xmt, srcs_m2r, dst_m2r, z_r)

        wl_r = params[f"W_l_m2r_{l}"]
        bl_r = params[f"b_l_m2r_{l}"].reshape(1, H)
        wr_r = params[f"W_r_m2r_{l}"]
        if l == NUM_LAYERS - 1:
            wo = jnp.zeros((H, H), F32).at[:OUT].set(params["W_out"])
            bo = jnp.zeros((1, H), F32).at[0, :OUT].set(params["b_out"])
            xr_new = _dense_r_final(agg_r, cnt_r, xr, wl_r, bl_r, wr_r, wo, bo)
        else:
            xr_new = _dense_r(agg_r, cnt_r, xr, wl_r, bl_r, wr_r)

        if l < NUM_LAYERS - 1:
            xrt = _split_cols(xr, 16)
            agg_m = _sc_r2m(xrt, srcs_r2m, dst_r2m, z_m)
            xm = _dense_m(agg_m, cnt_m, xm,
                          params[f"W_l_r2m_{l}"],
                          params[f"b_l_r2m_{l}"].reshape(1, H),
                          params[f"W_r_r2m_{l}"])
        xr = xr_new

    return xr[:N_REACT, :OUT]


# speedup vs baseline: 1.6038x; 1.6038x over previous
import functools

import jax
import jax.numpy as jnp
from jax import lax
from jax.experimental import pallas as pl
from jax.experimental.pallas import tpu as pltpu
from jax.experimental.pallas import tpu_sc as plsc

F32 = jnp.float32
I32 = jnp.int32

H = 128
OUT = 10
NUM_LAYERS = 4
N_MOL = 50000
N_REACT = 10000
E = 320000

NC = 2
NS = 16
NW = NC * NS

NM_P = 51200
NR_P = 10240
CHUNK = 128
E_PAD = 327680
EROWS = E_PAD // CHUNK
TCH = EROWS // NW
NPASS = 8

R_STRIPE = NR_P // NS
M_STRIPE = NM_P // NS

RING = 4

_SC_MESH = plsc.VectorSubcoreMesh(core_axis_name="c", subcore_axis_name="s")
_SC_PARAMS = pltpu.CompilerParams(use_tc_tiling_on_sc=False)


def _worker_base():
    c = lax.axis_index("c")
    s = lax.axis_index("s")
    return c, s, (s * NC + c) * TCH


def _ring_pass(tab, src_v, dst_v, bufs, sems, acc):
    def fire(j, k):
        return pltpu.async_copy(tab.at[src_v.at[j]], bufs[k], sems[k])

    for k in range(RING):
        fire(k, k)

    def group(g, carry):
        for k in range(RING):
            j = g * RING + k
            pltpu.make_async_copy(tab.at[src_v.at[j]], bufs[k], sems[k]).wait()
            pltpu.sync_copy(bufs[k], acc.at[dst_v.at[j]], add=True)

            @pl.when(j + RING < TCH)
            def _():
                fire(j + RING, k)
        return carry

    lax.fori_loop(0, TCH // RING, group, 0)


def _agg_body(n_tab, t_stripe, a_stripe, with_cnt,
              xt_hbm, src_hbm, dst_hbm, z_hbm, *rest):
    if with_cnt:
        (ones_hbm, zc_hbm, out_hbm, cnt_hbm, src_v, dst_v,
         ones_v, zb, tab, acc, cacc, *sems) = rest
    else:
        (out_hbm, src_v, dst_v, zb, tab, acc, *sems) = rest
    bufs = sems[:RING]
    sems = sems[RING:]
    c, s, base = _worker_base()
    pltpu.sync_copy(src_hbm.at[pl.ds(base, TCH)], src_v)
    pltpu.sync_copy(dst_hbm.at[pl.ds(base, TCH)], dst_v)
    pltpu.sync_copy(z_hbm, zb)
    t0 = s * t_stripe
    r0 = s * a_stripe
    if with_cnt:
        pltpu.sync_copy(ones_hbm, ones_v)
        pltpu.sync_copy(zc_hbm, cacc.at[pl.ds(r0, a_stripe)])
    for p in range(NPASS):
        pltpu.sync_copy(xt_hbm.at[pl.ds(p * n_tab + t0, t_stripe)],
                        tab.at[pl.ds(t0, t_stripe)])
        for q in range(a_stripe // zb.shape[0]):
            pltpu.sync_copy(zb, acc.at[pl.ds(r0 + q * zb.shape[0],
                                             zb.shape[0])])
        plsc.subcore_barrier()
        _ring_pass(tab, src_v, dst_v, bufs, sems, acc)
        if with_cnt and p == 0:
            cs = sems[RING]

            def cfire(g, carry):
                for k in range(4):
                    j = g * 4 + k
                    pltpu.async_copy(ones_v, cacc.at[dst_v.at[j]], cs,
                                     add=True)
                return carry

            lax.fori_loop(0, TCH // 4, cfire, 0)

            def cdrain(g, carry):
                for k in range(4):
                    j = g * 4 + k
                    pltpu.make_async_copy(ones_v, cacc.at[dst_v.at[j]],
                                          cs).wait()
                return carry

            lax.fori_loop(0, TCH // 4, cdrain, 0)
        plsc.subcore_barrier()
        pltpu.sync_copy(acc.at[pl.ds(r0, a_stripe)],
                        out_hbm.at[c, p, pl.ds(r0, a_stripe)])
    if with_cnt:
        pltpu.sync_copy(cacc.at[pl.ds(r0, a_stripe)],
                        cnt_hbm.at[c, pl.ds(r0, a_stripe)])


def _make_agg(n_tab, n_acc, with_cnt):
    t_stripe = n_tab // NS
    a_stripe = n_acc // NS
    zrows = min(a_stripe, 640)
    out_t = jax.ShapeDtypeStruct((NC, NPASS, n_acc, 16), F32)
    if with_cnt:
        out_t = (out_t, jax.ShapeDtypeStruct((NC, n_acc, 16), F32))
    scratch = [
        pltpu.VMEM((TCH, CHUNK), I32),
        pltpu.VMEM((TCH, CHUNK), I32),
    ]
    if with_cnt:
        scratch = scratch + [pltpu.VMEM((CHUNK, 16), F32)]
    scratch = scratch + [
        pltpu.VMEM((zrows, 16), F32),
        pltpu.VMEM_SHARED((n_tab, 16), F32),
        pltpu.VMEM_SHARED((n_acc, 16), F32),
    ]
    if with_cnt:
        scratch = scratch + [pltpu.VMEM_SHARED((n_acc, 16), F32)]
    scratch = scratch + [pltpu.VMEM((CHUNK, 16), F32)] * RING
    scratch = scratch + [pltpu.SemaphoreType.DMA] * (RING + (1 if with_cnt else 0))
    return functools.partial(
        pl.kernel,
        out_type=out_t,
        mesh=_SC_MESH,
        compiler_params=_SC_PARAMS,
        scratch_types=scratch,
    )(functools.partial(_agg_body, n_tab, t_stripe, a_stripe, with_cnt))


_sc_m2r_cnt = _make_agg(NM_P, NR_P, True)
_sc_m2r = _make_agg(NM_P, NR_P, False)
_sc_r2m = _make_agg(NR_P, NM_P, False)


@functools.partial(
    pl.kernel,
    out_type=jax.ShapeDtypeStruct((NC, NM_P, 16), F32),
    mesh=_SC_MESH,
    compiler_params=_SC_PARAMS,
    scratch_types=[
        pltpu.VMEM((TCH, CHUNK), I32),
        pltpu.VMEM((CHUNK, 16), F32),
        pltpu.VMEM_SHARED((NM_P, 16), F32),
        pltpu.SemaphoreType.DMA,
    ],
)
def _sc_cnt_m(dst_hbm, ones_hbm, zc_hbm, out_hbm, dst_v, ones_v, acc, s0):
    c, s, base = _worker_base()
    r0 = s * M_STRIPE
    pltpu.sync_copy(dst_hbm.at[pl.ds(base, TCH)], dst_v)
    pltpu.sync_copy(ones_hbm, ones_v)
    pltpu.sync_copy(zc_hbm, acc.at[pl.ds(r0, M_STRIPE)])
    plsc.subcore_barrier()

    def cfire(g, carry):
        for k in range(4):
            j = g * 4 + k
            pltpu.async_copy(ones_v, acc.at[dst_v.at[j]], s0, add=True)
        return carry

    lax.fori_loop(0, TCH // 4, cfire, 0)

    def cdrain(g, carry):
        for k in range(4):
            j = g * 4 + k
            pltpu.make_async_copy(ones_v, acc.at[dst_v.at[j]], s0).wait()
        return carry

    lax.fori_loop(0, TCH // 4, cdrain, 0)
    plsc.subcore_barrier()
    pltpu.sync_copy(acc.at[pl.ds(r0, M_STRIPE)],
                    out_hbm.at[c, pl.ds(r0, M_STRIPE)])


def _dot_t(a, b):
    return lax.dot_general(a, b, (((1,), (1,)), ((), ())),
                           preferred_element_type=F32)


def _merge_agg(agg_ref, cnt_ref):
    parts = [agg_ref[0, p] + agg_ref[1, p] for p in range(NPASS)]
    ssum = jnp.concatenate(parts, axis=1)
    cnt = cnt_ref[0][:, 0:1] + cnt_ref[1][:, 0:1]
    return ssum * (1.0 / jnp.maximum(cnt, 1.0))


def _dense_body(final, agg_ref, cnt_ref, x_ref, wl_ref, bl_ref, wr_ref,
                *rest):
    if final:
        wo_ref, bo_ref, o_ref = rest
    else:
        (o_ref,) = rest
    agg = _merge_agg(agg_ref, cnt_ref)
    h = _dot_t(agg, wl_ref[...]) + bl_ref[...] + _dot_t(x_ref[...], wr_ref[...])
    r = jnp.maximum(h, 0.0)
    if final:
        o_ref[...] = _dot_t(r, wo_ref[...]) + bo_ref[...]
    else:
        o_ref[...] = r


def _make_dense(nrows, final):
    blk = 1024
    grid = nrows // blk
    full = lambda shape: pl.BlockSpec(shape, lambda i: (0,) * len(shape))
    in_specs = [
        pl.BlockSpec((NC, NPASS, blk, 16), lambda i: (0, 0, i, 0)),
        pl.BlockSpec((NC, blk, 16), lambda i: (0, i, 0)),
        pl.BlockSpec((blk, H), lambda i: (i, 0)),
        full((H, H)), full((1, H)), full((H, H)),
    ]
    if final:
        in_specs += [full((H, H)), full((1, H))]
    return pl.pallas_call(
        functools.partial(_dense_body, final),
        grid=(grid,),
        in_specs=in_specs,
        out_specs=pl.BlockSpec((blk, H), lambda i: (i, 0)),
        out_shape=jax.ShapeDtypeStruct((nrows, H), F32),
    )


_dense_r = _make_dense(NR_P, False)
_dense_r_final = _make_dense(NR_P, True)
_dense_m = _make_dense(NM_P, False)


def _pad_edges(row, fill):
    row = row.astype(I32)
    return jnp.concatenate(
        [row, jnp.full((E_PAD - E,), fill, dtype=I32)]).reshape(EROWS, CHUNK)


def _split_cols(x, width=16):
    n, d = x.shape
    k = d // width
    return x.reshape(n, k, width).transpose(1, 0, 2).reshape(k * n, width)


def kernel(x_molecule, x_reaction, edge_index_m2r, edge_index_r2m, params):
    xm = jnp.pad(x_molecule, ((0, NM_P - N_MOL), (0, 0)))
    xr = jnp.pad(x_reaction, ((0, NR_P - N_REACT), (0, 0)))

    src_m2r = _pad_edges(edge_index_m2r[0], 0)
    dst_m2r = _pad_edges(edge_index_m2r[1], N_REACT)
    dst_r2m = _pad_edges(edge_index_r2m[1], N_MOL)
    src_r2m = _pad_edges(edge_index_r2m[0], 0)

    ones16 = jnp.ones((CHUNK, 16), F32)
    z_r = jnp.zeros((R_STRIPE, 16), F32)
    z_m = jnp.zeros((640, 16), F32)
    z_mc = jnp.zeros((M_STRIPE, 16), F32)

    cnt_m = _sc_cnt_m(dst_r2m, ones16, z_mc)

    cnt_r = None
    for l in range(NUM_LAYERS):
        xmt = _split_cols(xm)
        if l == 0:
            agg_r, cnt_r = _sc_m2r_cnt(xmt, src_m2r, dst_m2r, z_r,
                                       ones16, z_r)
        else:
            agg_r = _sc_m2r(xmt, src_m2r, dst_m2r, z_r)

        wl_r = params[f"W_l_m2r_{l}"]
        bl_r = params[f"b_l_m2r_{l}"].reshape(1, H)
        wr_r = params[f"W_r_m2r_{l}"]
        if l == NUM_LAYERS - 1:
            wo = jnp.zeros((H, H), F32).at[:OUT].set(params["W_out"])
            bo = jnp.zeros((1, H), F32).at[0, :OUT].set(params["b_out"])
            xr_new = _dense_r_final(agg_r, cnt_r, xr, wl_r, bl_r, wr_r, wo, bo)
        else:
            xr_new = _dense_r(agg_r, cnt_r, xr, wl_r, bl_r, wr_r)

        if l < NUM_LAYERS - 1:
            xrt = _split_cols(xr)
            agg_m = _sc_r2m(xrt, src_r2m, dst_r2m, z_m)
            xm = _dense_m(agg_m, cnt_m, xm,
                          params[f"W_l_r2m_{l}"],
                          params[f"b_l_r2m_{l}"].reshape(1, H),
                          params[f"W_r_r2m_{l}"])
        xr = xr_new

    return xr[:N_REACT, :OUT]


# speedup vs baseline: 2.4774x; 1.5447x over previous
import functools

import jax
import jax.numpy as jnp
from jax import lax
from jax.experimental import pallas as pl
from jax.experimental.pallas import tpu as pltpu
from jax.experimental.pallas import tpu_sc as plsc

F32 = jnp.float32
I32 = jnp.int32

H = 128
OUT = 10
NUM_LAYERS = 4
N_MOL = 50000
N_REACT = 10000
E = 320000

NC = 2
NS = 16
NW = NC * NS

NM_P = 51200
NR_P = 10240
CHUNK = 128
E_PAD = 327680
EROWS = E_PAD // CHUNK
TCH = EROWS // NW
NPASS = 8

R_STRIPE = NR_P // NS
M_STRIPE = NM_P // NS

RING = 4

_SC_MESH = plsc.VectorSubcoreMesh(core_axis_name="c", subcore_axis_name="s")
_SC_PARAMS = pltpu.CompilerParams(use_tc_tiling_on_sc=False)


def _worker_base():
    c = lax.axis_index("c")
    s = lax.axis_index("s")
    return c, s, (s * NC + c) * TCH


def _ring_pass(tab, src_v, dst_v, bufs, sems, acc):
    def fire(j, k):
        return pltpu.async_copy(tab.at[src_v.at[j]], bufs[k], sems[k])

    for k in range(RING):
        fire(k, k)

    def group(g, carry):
        for k in range(RING):
            j = g * RING + k
            pltpu.make_async_copy(tab.at[src_v.at[j]], bufs[k], sems[k]).wait()
            pltpu.sync_copy(bufs[k], acc.at[dst_v.at[j]], add=True)

            @pl.when(j + RING < TCH)
            def _():
                fire(j + RING, k)
        return carry

    lax.fori_loop(0, TCH // RING, group, 0)


def _agg_body(n_tab, t_stripe, a_stripe, with_cnt,
              xt_hbm, src_hbm, dst_hbm, z_hbm, *rest):
    if with_cnt:
        (ones_hbm, zc_hbm, out_hbm, cnt_hbm, src_v, dst_v,
         ones_v, zb, tab, acc, cacc, *sems) = rest
    else:
        (out_hbm, src_v, dst_v, zb, tab, acc, *sems) = rest
    bufs = sems[:RING]
    sems = sems[RING:]
    c, s, base = _worker_base()
    pltpu.sync_copy(src_hbm.at[pl.ds(base, TCH)], src_v)
    pltpu.sync_copy(dst_hbm.at[pl.ds(base, TCH)], dst_v)
    pltpu.sync_copy(z_hbm, zb)
    t0 = s * t_stripe
    r0 = s * a_stripe
    if with_cnt:
        pltpu.sync_copy(ones_hbm, ones_v)
        pltpu.sync_copy(zc_hbm, cacc.at[pl.ds(r0, a_stripe)])
    for p in range(NPASS):
        pltpu.sync_copy(xt_hbm.at[pl.ds(p * n_tab + t0, t_stripe)],
                        tab.at[pl.ds(t0, t_stripe)])
        for q in range(a_stripe // zb.shape[0]):
            pltpu.sync_copy(zb, acc.at[pl.ds(r0 + q * zb.shape[0],
                                             zb.shape[0])])
        plsc.subcore_barrier()
        _ring_pass(tab, src_v, dst_v, bufs, sems, acc)
        if with_cnt and p == 0:
            cs = sems[RING]

            def cfire(g, carry):
                for k in range(4):
                    j = g * 4 + k
                    pltpu.async_copy(ones_v, cacc.at[dst_v.at[j]], cs,
                                     add=True)
                return carry

            lax.fori_loop(0, TCH // 4, cfire, 0)

            def cdrain(g, carry):
                for k in range(4):
                    j = g * 4 + k
                    pltpu.make_async_copy(ones_v, cacc.at[dst_v.at[j]],
                                          cs).wait()
                return carry

            lax.fori_loop(0, TCH // 4, cdrain, 0)
        plsc.subcore_barrier()
        pltpu.sync_copy(acc.at[pl.ds(r0, a_stripe)],
                        out_hbm.at[c, pl.ds(r0, a_stripe),
                                   pl.ds(p * 16, 16)])
    if with_cnt:
        pltpu.sync_copy(cacc.at[pl.ds(r0, a_stripe)],
                        cnt_hbm.at[c, pl.ds(r0, a_stripe)])


def _make_agg(n_tab, n_acc, with_cnt):
    t_stripe = n_tab // NS
    a_stripe = n_acc // NS
    zrows = min(a_stripe, 640)
    out_t = jax.ShapeDtypeStruct((NC, n_acc, H), F32)
    if with_cnt:
        out_t = (out_t, jax.ShapeDtypeStruct((NC, n_acc, 16), F32))
    scratch = [
        pltpu.VMEM((TCH, CHUNK), I32),
        pltpu.VMEM((TCH, CHUNK), I32),
    ]
    if with_cnt:
        scratch = scratch + [pltpu.VMEM((CHUNK, 16), F32)]
    scratch = scratch + [
        pltpu.VMEM((zrows, 16), F32),
        pltpu.VMEM_SHARED((n_tab, 16), F32),
        pltpu.VMEM_SHARED((n_acc, 16), F32),
    ]
    if with_cnt:
        scratch = scratch + [pltpu.VMEM_SHARED((n_acc, 16), F32)]
    scratch = scratch + [pltpu.VMEM((CHUNK, 16), F32)] * RING
    scratch = scratch + [pltpu.SemaphoreType.DMA] * (RING + (1 if with_cnt else 0))
    return functools.partial(
        pl.kernel,
        out_type=out_t,
        mesh=_SC_MESH,
        compiler_params=_SC_PARAMS,
        scratch_types=scratch,
    )(functools.partial(_agg_body, n_tab, t_stripe, a_stripe, with_cnt))


_sc_m2r_cnt = _make_agg(NM_P, NR_P, True)
_sc_m2r = _make_agg(NM_P, NR_P, False)
_sc_r2m = _make_agg(NR_P, NM_P, False)


@functools.partial(
    pl.kernel,
    out_type=jax.ShapeDtypeStruct((NC, NM_P, 16), F32),
    mesh=_SC_MESH,
    compiler_params=_SC_PARAMS,
    scratch_types=[
        pltpu.VMEM((TCH, CHUNK), I32),
        pltpu.VMEM((CHUNK, 16), F32),
        pltpu.VMEM_SHARED((NM_P, 16), F32),
        pltpu.SemaphoreType.DMA,
    ],
)
def _sc_cnt_m(dst_hbm, ones_hbm, zc_hbm, out_hbm, dst_v, ones_v, acc, s0):
    c, s, base = _worker_base()
    r0 = s * M_STRIPE
    pltpu.sync_copy(dst_hbm.at[pl.ds(base, TCH)], dst_v)
    pltpu.sync_copy(ones_hbm, ones_v)
    pltpu.sync_copy(zc_hbm, acc.at[pl.ds(r0, M_STRIPE)])
    plsc.subcore_barrier()

    def cfire(g, carry):
        for k in range(4):
            j = g * 4 + k
            pltpu.async_copy(ones_v, acc.at[dst_v.at[j]], s0, add=True)
        return carry

    lax.fori_loop(0, TCH // 4, cfire, 0)

    def cdrain(g, carry):
        for k in range(4):
            j = g * 4 + k
            pltpu.make_async_copy(ones_v, acc.at[dst_v.at[j]], s0).wait()
        return carry

    lax.fori_loop(0, TCH // 4, cdrain, 0)
    plsc.subcore_barrier()
    pltpu.sync_copy(acc.at[pl.ds(r0, M_STRIPE)],
                    out_hbm.at[c, pl.ds(r0, M_STRIPE)])


def _dot_t(a, b):
    return lax.dot_general(a, b, (((1,), (1,)), ((), ())),
                           preferred_element_type=F32)


def _dense_body(final, emit_t, agg_ref, cnt_ref, x_ref, wl_ref, bl_ref,
                wr_ref, *rest):
    if final:
        wo_ref, bo_ref, o_ref = rest
    elif emit_t:
        o_ref, t_ref = rest
    else:
        (o_ref,) = rest
    ssum = agg_ref[0] + agg_ref[1]
    cnt = cnt_ref[0][:, 0:1] + cnt_ref[1][:, 0:1]
    agg = ssum * (1.0 / jnp.maximum(cnt, 1.0))
    h = _dot_t(agg, wl_ref[...]) + bl_ref[...] + _dot_t(x_ref[...], wr_ref[...])
    r = jnp.maximum(h, 0.0)
    if final:
        o_ref[...] = _dot_t(r, wo_ref[...]) + bo_ref[...]
    else:
        o_ref[...] = r
        if emit_t:
            for p in range(NPASS):
                t_ref[p] = r[:, p * 16:(p + 1) * 16]


def _make_dense(nrows, final=False, emit_t=False):
    blk = 1024
    grid = nrows // blk
    full = lambda shape: pl.BlockSpec(shape, lambda i: (0,) * len(shape))
    in_specs = [
        pl.BlockSpec((NC, blk, H), lambda i: (0, i, 0)),
        pl.BlockSpec((NC, blk, 16), lambda i: (0, i, 0)),
        pl.BlockSpec((blk, H), lambda i: (i, 0)),
        full((H, H)), full((1, H)), full((H, H)),
    ]
    if final:
        in_specs += [full((H, H)), full((1, H))]
    out_specs = pl.BlockSpec((blk, H), lambda i: (i, 0))
    out_shape = jax.ShapeDtypeStruct((nrows, H), F32)
    if emit_t:
        out_specs = (out_specs,
                     pl.BlockSpec((NPASS, blk, 16), lambda i: (0, i, 0)))
        out_shape = (out_shape,
                     jax.ShapeDtypeStruct((NPASS, nrows, 16), F32))
    return pl.pallas_call(
        functools.partial(_dense_body, final, emit_t),
        grid=(grid,),
        in_specs=in_specs,
        out_specs=out_specs,
        out_shape=out_shape,
    )


_dense_r_t = _make_dense(NR_P, emit_t=True)
_dense_r = _make_dense(NR_P)
_dense_r_final = _make_dense(NR_P, final=True)
_dense_m_t = _make_dense(NM_P, emit_t=True)


def _pad_edges(row, fill):
    row = row.astype(I32)
    return jnp.concatenate(
        [row, jnp.full((E_PAD - E,), fill, dtype=I32)]).reshape(EROWS, CHUNK)


def _split_cols(x, width=16):
    n, d = x.shape
    k = d // width
    return x.reshape(n, k, width).transpose(1, 0, 2).reshape(k * n, width)


def kernel(x_molecule, x_reaction, edge_index_m2r, edge_index_r2m, params):
    xm = jnp.pad(x_molecule, ((0, NM_P - N_MOL), (0, 0)))
    xr = jnp.pad(x_reaction, ((0, NR_P - N_REACT), (0, 0)))

    src_m2r = _pad_edges(edge_index_m2r[0], 0)
    dst_m2r = _pad_edges(edge_index_m2r[1], N_REACT)
    dst_r2m = _pad_edges(edge_index_r2m[1], N_MOL)
    src_r2m = _pad_edges(edge_index_r2m[0], 0)

    ones16 = jnp.ones((CHUNK, 16), F32)
    z_r = jnp.zeros((R_STRIPE, 16), F32)
    z_m = jnp.zeros((640, 16), F32)
    z_mc = jnp.zeros((M_STRIPE, 16), F32)

    cnt_m = _sc_cnt_m(dst_r2m, ones16, z_mc)

    xmt = _split_cols(xm)
    xrt = _split_cols(xr)
    cnt_r = None
    for l in range(NUM_LAYERS):
        if l == 0:
            agg_r, cnt_r = _sc_m2r_cnt(xmt, src_m2r, dst_m2r, z_r,
                                       ones16, z_r)
        else:
            agg_r = _sc_m2r(xmt, src_m2r, dst_m2r, z_r)

        wl_r = params[f"W_l_m2r_{l}"]
        bl_r = params[f"b_l_m2r_{l}"].reshape(1, H)
        wr_r = params[f"W_r_m2r_{l}"]
        if l == NUM_LAYERS - 1:
            wo = jnp.zeros((H, H), F32).at[:OUT].set(params["W_out"])
            bo = jnp.zeros((1, H), F32).at[0, :OUT].set(params["b_out"])
            xr_new = _dense_r_final(agg_r, cnt_r, xr, wl_r, bl_r, wr_r, wo, bo)
            xrt_new = None
        elif l < NUM_LAYERS - 2:
            xr_new, xrt_new = _dense_r_t(agg_r, cnt_r, xr, wl_r, bl_r, wr_r)
            xrt_new = xrt_new.reshape(NPASS * NR_P, 16)
        else:
            xr_new = _dense_r(agg_r, cnt_r, xr, wl_r, bl_r, wr_r)
            xrt_new = None

        if l < NUM_LAYERS - 1:
            agg_m = _sc_r2m(xrt, src_r2m, dst_r2m, z_m)
            xm, xmt = _dense_m_t(agg_m, cnt_m, xm,
                                 params[f"W_l_r2m_{l}"],
                                 params[f"b_l_r2m_{l}"].reshape(1, H),
                                 params[f"W_r_r2m_{l}"])
            xmt = xmt.reshape(NPASS * NM_P, 16)
        xr = xr_new
        xrt = xrt_new

    return xr[:N_REACT, :OUT]
